# Initial kernel scaffold; baseline (speedup 1.0000x reference)
import jax
import jax.numpy as jnp
from jax.experimental import pallas as pl


def kernel(x):
    raise NotImplementedError("write your pallas kernel here")


# speedup vs baseline: 32.9364x; 32.9364x over previous
import functools

import jax
import jax.numpy as jnp
import numpy as np
from jax import lax
from jax.experimental import pallas as pl
from jax.experimental.pallas import tpu as pltpu
from jax.experimental.pallas import tpu_sc as plsc

BINS = 2048
Q_MIN, Q_MAX = 0, 255
EPS = float(np.finfo(np.float32).eps)

N = 33554432
ROWS, COLS = 8192, 4096
MM_BLK = 512
FQ_BLK = 256

NC, NS, L = 2, 16, 16
NW = NC * NS
PER_W = N // NW
CHUNK = 16384
NCH = PER_W // CHUNK
HIST_WORDS = L * BINS


def _mm_body(x_ref, min_ref, max_ref):
    i = pl.program_id(0)
    bmin = jnp.min(x_ref[...])
    bmax = jnp.max(x_ref[...])

    @pl.when(i == 0)
    def _():
        min_ref[0, 0] = bmin
        max_ref[0, 0] = bmax

    @pl.when(i != 0)
    def _():
        min_ref[0, 0] = jnp.minimum(min_ref[0, 0], bmin)
        max_ref[0, 0] = jnp.maximum(max_ref[0, 0], bmax)


_minmax = pl.pallas_call(
    _mm_body,
    grid=(ROWS // MM_BLK,),
    in_specs=[pl.BlockSpec((MM_BLK, COLS), lambda i: (i, 0))],
    out_specs=[
        pl.BlockSpec((1, 1), lambda i: (0, 0), memory_space=pltpu.SMEM),
        pl.BlockSpec((1, 1), lambda i: (0, 0), memory_space=pltpu.SMEM),
    ],
    out_shape=[
        jax.ShapeDtypeStruct((1, 1), jnp.float32),
        jax.ShapeDtypeStruct((1, 1), jnp.float32),
    ],
)


def _fq_body(qp_ref, x_ref, o_ref):
    s = qp_ref[0, 0]
    inv_s = qp_ref[0, 1]
    z = qp_ref[0, 2]
    q = jnp.clip(jnp.round(x_ref[...] * inv_s) + z, 0.0, 255.0)
    o_ref[...] = (q - z) * s


_fakequant = pl.pallas_call(
    _fq_body,
    grid=(ROWS // FQ_BLK,),
    in_specs=[
        pl.BlockSpec(memory_space=pltpu.SMEM),
        pl.BlockSpec((FQ_BLK, COLS), lambda i: (i, 0)),
    ],
    out_specs=pl.BlockSpec((FQ_BLK, COLS), lambda i: (i, 0)),
    out_shape=jax.ShapeDtypeStruct((ROWS, COLS), jnp.float32),
)


def _hsum_body(h_ref, o_ref):
    o_ref[...] = jnp.sum(h_ref[...], axis=0, keepdims=True)


_hist_sum = pl.pallas_call(
    _hsum_body,
    out_shape=jax.ShapeDtypeStruct((1, BINS), jnp.float32),
)


@functools.partial(
    pl.kernel,
    out_type=jax.ShapeDtypeStruct((NW * BINS,), jnp.float32),
    mesh=plsc.VectorSubcoreMesh(core_axis_name="c", subcore_axis_name="s"),
    compiler_params=pltpu.CompilerParams(needs_layout_passes=False),
    scratch_types=[
        pltpu.VMEM((CHUNK,), jnp.float32),
        pltpu.VMEM((HIST_WORDS,), jnp.float32),
        pltpu.VMEM((BINS,), jnp.float32),
        pltpu.VMEM((L,), jnp.float32),
        pltpu.VMEM((L,), jnp.float32),
    ],
)
def _sc_hist(x_hbm, params_hbm, out_hbm, buf, hist, red, pmin, pinvw):
    wid = lax.axis_index("s") * NC + lax.axis_index("c")
    base = wid * PER_W

    pltpu.sync_copy(params_hbm.at[pl.ds(0, L)], pmin)
    pltpu.sync_copy(params_hbm.at[pl.ds(L, L)], pinvw)
    minv = pmin[...]
    invw = pinvw[...]
    lane_off = lax.iota(jnp.int32, L) * BINS
    ones = jnp.full((L,), 1.0, jnp.float32)
    zeros = jnp.zeros((L,), jnp.float32)

    def zero_body(j, _):
        hist[pl.ds(j * L, L)] = zeros
        return 0

    lax.fori_loop(0, HIST_WORDS // L, zero_body, 0)

    def chunk_body(c, _):
        pltpu.sync_copy(x_hbm.at[pl.ds(base + c * CHUNK, CHUNK)], buf)

        def vec_body(j, _):
            xv = buf[pl.ds(j * L, L)]
            t = (xv - minv) * invw
            bi = jnp.minimum(t.astype(jnp.int32), BINS - 1)
            plsc.addupdate_scatter(hist, [bi + lane_off], ones)
            return 0

        lax.fori_loop(0, CHUNK // L, vec_body, 0, unroll=4)
        return 0

    lax.fori_loop(0, NCH, chunk_body, 0)

    def red_body(j, _):
        acc = hist[pl.ds(j * L, L)]
        for l in range(1, L):
            acc = acc + hist[pl.ds(l * BINS + j * L, L)]
        red[pl.ds(j * L, L)] = acc
        return 0

    lax.fori_loop(0, BINS // L, red_body, 0)
    pltpu.sync_copy(red, out_hbm.at[pl.ds(wid * BINS, BINS)])


def kernel(x):
    xr = x.reshape(ROWS, COLS)
    mn, mx = _minmax(xr)
    min_val = mn[0, 0]
    max_val = mx[0, 0]

    bin_width = (max_val - min_val) / BINS
    safe_w = jnp.maximum(bin_width, EPS)
    inv_w = 1.0 / safe_w

    min_neg = jnp.minimum(min_val, 0.0)
    max_pos = jnp.maximum(max_val, 0.0)
    scale = jnp.maximum((max_pos - min_neg) / float(Q_MAX - Q_MIN), EPS)
    zero_point = jnp.clip(Q_MIN - jnp.round(min_neg / scale),
                          float(Q_MIN), float(Q_MAX))

    params = jnp.concatenate([
        jnp.broadcast_to(min_val, (L,)),
        jnp.broadcast_to(inv_w, (L,)),
    ]).astype(jnp.float32)
    hist32 = _sc_hist(x, params)
    histogram = _hist_sum(hist32.reshape(NW, BINS)).reshape(BINS)

    qp = jnp.stack([scale, 1.0 / scale, zero_point]).reshape(1, 3)
    out = _fakequant(qp, xr).reshape(N)

    return out, histogram, scale, zero_point.astype(jnp.int32)


# speedup vs baseline: 89.9916x; 2.7323x over previous
import functools

import jax
import jax.numpy as jnp
import numpy as np
from jax import lax
from jax.experimental import pallas as pl
from jax.experimental.pallas import tpu as pltpu
from jax.experimental.pallas import tpu_sc as plsc

BINS = 2048
Q_MIN, Q_MAX = 0, 255
EPS = float(np.finfo(np.float32).eps)

N = 33554432
ROWS, COLS = 8192, 4096
MM_BLK = 512
FQ_BLK = 256

NC, NS, L = 2, 16, 16
NW = NC * NS
PER_W = N // NW
CHUNK = 16384
NCH = PER_W // CHUNK
HIST_WORDS = L * BINS


def _mm_body(x_ref, min_ref, max_ref):
    i = pl.program_id(0)
    bmin = jnp.min(x_ref[...])
    bmax = jnp.max(x_ref[...])

    @pl.when(i == 0)
    def _():
        min_ref[0, 0] = bmin
        max_ref[0, 0] = bmax

    @pl.when(i != 0)
    def _():
        min_ref[0, 0] = jnp.minimum(min_ref[0, 0], bmin)
        max_ref[0, 0] = jnp.maximum(max_ref[0, 0], bmax)


_minmax = pl.pallas_call(
    _mm_body,
    grid=(ROWS // MM_BLK,),
    in_specs=[pl.BlockSpec((MM_BLK, COLS), lambda i: (i, 0))],
    out_specs=[
        pl.BlockSpec((1, 1), lambda i: (0, 0), memory_space=pltpu.SMEM),
        pl.BlockSpec((1, 1), lambda i: (0, 0), memory_space=pltpu.SMEM),
    ],
    out_shape=[
        jax.ShapeDtypeStruct((1, 1), jnp.float32),
        jax.ShapeDtypeStruct((1, 1), jnp.float32),
    ],
)


def _fq_body(qp_ref, x_ref, o_ref):
    s = qp_ref[0, 0]
    inv_s = qp_ref[0, 1]
    z = qp_ref[0, 2]
    q = jnp.clip(jnp.round(x_ref[...] * inv_s) + z, 0.0, 255.0)
    o_ref[...] = (q - z) * s


_fakequant = pl.pallas_call(
    _fq_body,
    grid=(ROWS // FQ_BLK,),
    in_specs=[
        pl.BlockSpec(memory_space=pltpu.SMEM),
        pl.BlockSpec((FQ_BLK, COLS), lambda i: (i, 0)),
    ],
    out_specs=pl.BlockSpec((FQ_BLK, COLS), lambda i: (i, 0)),
    out_shape=jax.ShapeDtypeStruct((ROWS, COLS), jnp.float32),
)


def _hsum_body(h_ref, o_ref):
    o_ref[...] = jnp.sum(h_ref[...], axis=0, keepdims=True)


_hist_sum = pl.pallas_call(
    _hsum_body,
    out_shape=jax.ShapeDtypeStruct((1, BINS), jnp.float32),
)


@functools.partial(
    pl.kernel,
    out_type=jax.ShapeDtypeStruct((NW * BINS,), jnp.float32),
    mesh=plsc.VectorSubcoreMesh(core_axis_name="c", subcore_axis_name="s"),
    compiler_params=pltpu.CompilerParams(needs_layout_passes=False),
    scratch_types=[
        pltpu.VMEM((CHUNK,), jnp.float32),
        pltpu.VMEM((CHUNK,), jnp.float32),
        pltpu.VMEM((HIST_WORDS,), jnp.float32),
        pltpu.VMEM((BINS,), jnp.float32),
        pltpu.VMEM((L,), jnp.float32),
        pltpu.VMEM((L,), jnp.float32),
        pltpu.SemaphoreType.DMA,
        pltpu.SemaphoreType.DMA,
    ],
)
def _sc_hist(x_hbm, params_hbm, out_hbm, buf_a, buf_b, hist, red,
             pinvw, pbias, sem_a, sem_b):
    wid = lax.axis_index("s") * NC + lax.axis_index("c")
    base = wid * PER_W

    pltpu.sync_copy(params_hbm.at[pl.ds(0, L)], pinvw)
    pltpu.sync_copy(params_hbm.at[pl.ds(L, L)], pbias)
    invw = pinvw[...]
    bias = pbias[...]
    lane_off = lax.iota(jnp.int32, L) * BINS
    ones = jnp.full((L,), 1.0, jnp.float32)
    zeros = jnp.zeros((L,), jnp.float32)

    def zero_body(j, _):
        hist[pl.ds(j * L, L)] = zeros
        return 0

    lax.fori_loop(0, HIST_WORDS // L, zero_body, 0)

    def process(buf):
        @plsc.parallel_loop(0, CHUNK // L, unroll=8)
        def _(j):
            xv = buf[pl.ds(j * L, L)]
            t = xv * invw + bias
            bi = jnp.minimum(t.astype(jnp.int32), BINS - 1)
            plsc.addupdate_scatter(hist, [bi + lane_off], ones)

    pltpu.async_copy(x_hbm.at[pl.ds(base, CHUNK)], buf_a, sem_a)

    def pair_body(p, _):
        c0 = 2 * p
        pltpu.make_async_copy(
            x_hbm.at[pl.ds(base + c0 * CHUNK, CHUNK)], buf_a, sem_a).wait()
        pltpu.async_copy(
            x_hbm.at[pl.ds(base + (c0 + 1) * CHUNK, CHUNK)], buf_b, sem_b)
        process(buf_a)
        pltpu.make_async_copy(
            x_hbm.at[pl.ds(base + (c0 + 1) * CHUNK, CHUNK)], buf_b, sem_b).wait()

        @pl.when(c0 + 2 < NCH)
        def _():
            pltpu.async_copy(
                x_hbm.at[pl.ds(base + (c0 + 2) * CHUNK, CHUNK)], buf_a, sem_a)

        process(buf_b)
        return 0

    lax.fori_loop(0, NCH // 2, pair_body, 0)

    def red_body(j, _):
        acc = hist[pl.ds(j * L, L)]
        for l in range(1, L):
            acc = acc + hist[pl.ds(l * BINS + j * L, L)]
        red[pl.ds(j * L, L)] = acc
        return 0

    lax.fori_loop(0, BINS // L, red_body, 0)
    pltpu.sync_copy(red, out_hbm.at[pl.ds(wid * BINS, BINS)])


def kernel(x):
    xr = x.reshape(ROWS, COLS)
    mn, mx = _minmax(xr)
    min_val = mn[0, 0]
    max_val = mx[0, 0]

    bin_width = (max_val - min_val) / BINS
    safe_w = jnp.maximum(bin_width, EPS)
    inv_w = 1.0 / safe_w

    min_neg = jnp.minimum(min_val, 0.0)
    max_pos = jnp.maximum(max_val, 0.0)
    scale = jnp.maximum((max_pos - min_neg) / float(Q_MAX - Q_MIN), EPS)
    zero_point = jnp.clip(Q_MIN - jnp.round(min_neg / scale),
                          float(Q_MIN), float(Q_MAX))

    params = jnp.concatenate([
        jnp.broadcast_to(inv_w, (L,)),
        jnp.broadcast_to(-min_val * inv_w, (L,)),
    ]).astype(jnp.float32)
    hist32 = _sc_hist(x, params)
    histogram = _hist_sum(hist32.reshape(NW, BINS)).reshape(BINS)

    qp = jnp.stack([scale, 1.0 / scale, zero_point]).reshape(1, 3)
    out = _fakequant(qp, xr).reshape(N)

    return out, histogram, scale, zero_point.astype(jnp.int32)


# speedup vs baseline: 90.0525x; 1.0007x over previous
import functools

import jax
import jax.numpy as jnp
import numpy as np
from jax import lax
from jax.experimental import pallas as pl
from jax.experimental.pallas import tpu as pltpu
from jax.experimental.pallas import tpu_sc as plsc

BINS = 2048
Q_MIN, Q_MAX = 0, 255
EPS = float(np.finfo(np.float32).eps)

N = 33554432
ROWS, COLS = 8192, 4096
MM_BLK = 512
FQ_BLK = 256

NC, NS, L = 2, 16, 16
NW = NC * NS
PER_W = N // NW
CHUNK = 16384
NCH = PER_W // CHUNK
HIST_WORDS = L * BINS


def _mm_body(x_ref, min_ref, max_ref):
    i = pl.program_id(0)
    bmin = jnp.min(x_ref[...])
    bmax = jnp.max(x_ref[...])

    @pl.when(i == 0)
    def _():
        min_ref[0, 0] = bmin
        max_ref[0, 0] = bmax

    @pl.when(i != 0)
    def _():
        min_ref[0, 0] = jnp.minimum(min_ref[0, 0], bmin)
        max_ref[0, 0] = jnp.maximum(max_ref[0, 0], bmax)


_minmax = pl.pallas_call(
    _mm_body,
    grid=(ROWS // MM_BLK,),
    in_specs=[pl.BlockSpec((MM_BLK, COLS), lambda i: (i, 0))],
    out_specs=[
        pl.BlockSpec((1, 1), lambda i: (0, 0), memory_space=pltpu.SMEM),
        pl.BlockSpec((1, 1), lambda i: (0, 0), memory_space=pltpu.SMEM),
    ],
    out_shape=[
        jax.ShapeDtypeStruct((1, 1), jnp.float32),
        jax.ShapeDtypeStruct((1, 1), jnp.float32),
    ],
)


def _fq_body(qp_ref, x_ref, o_ref):
    s = qp_ref[0, 0]
    inv_s = qp_ref[0, 1]
    z = qp_ref[0, 2]
    q = jnp.clip(jnp.round(x_ref[...] * inv_s) + z, 0.0, 255.0)
    o_ref[...] = (q - z) * s


_fakequant = pl.pallas_call(
    _fq_body,
    grid=(ROWS // FQ_BLK,),
    in_specs=[
        pl.BlockSpec(memory_space=pltpu.SMEM),
        pl.BlockSpec((FQ_BLK, COLS), lambda i: (i, 0)),
    ],
    out_specs=pl.BlockSpec((FQ_BLK, COLS), lambda i: (i, 0)),
    out_shape=jax.ShapeDtypeStruct((ROWS, COLS), jnp.float32),
)


def _hsum_body(h_ref, o_ref):
    o_ref[...] = jnp.sum(h_ref[...], axis=0, keepdims=True)


_hist_sum = pl.pallas_call(
    _hsum_body,
    out_shape=jax.ShapeDtypeStruct((1, BINS), jnp.float32),
)


@functools.partial(
    pl.kernel,
    out_type=jax.ShapeDtypeStruct((NW * BINS,), jnp.float32),
    mesh=plsc.VectorSubcoreMesh(core_axis_name="c", subcore_axis_name="s"),
    compiler_params=pltpu.CompilerParams(needs_layout_passes=False,
                                         use_tc_tiling_on_sc=True),
    scratch_types=[
        pltpu.VMEM((CHUNK,), jnp.float32),
        pltpu.VMEM((CHUNK,), jnp.float32),
        pltpu.VMEM((HIST_WORDS,), jnp.float32),
        pltpu.VMEM((BINS,), jnp.float32),
        pltpu.VMEM((L,), jnp.float32),
        pltpu.VMEM((L,), jnp.float32),
        pltpu.SemaphoreType.DMA,
        pltpu.SemaphoreType.DMA,
    ],
)
def _sc_hist(x_hbm, params_hbm, out_hbm, buf_a, buf_b, hist, red,
             pinvw, pbias, sem_a, sem_b):
    wid = lax.axis_index("s") * NC + lax.axis_index("c")
    base = wid * PER_W

    pltpu.sync_copy(params_hbm.at[pl.ds(0, L)], pinvw)
    pltpu.sync_copy(params_hbm.at[pl.ds(L, L)], pbias)
    invw = pinvw[...]
    bias = pbias[...]
    lane_off = lax.iota(jnp.int32, L) * BINS
    ones = jnp.full((L,), 1.0, jnp.float32)
    zeros = jnp.zeros((L,), jnp.float32)

    def zero_body(j, _):
        hist[pl.ds(j * L, L)] = zeros
        return 0

    lax.fori_loop(0, HIST_WORDS // L, zero_body, 0)

    def process(buf):
        @plsc.parallel_loop(0, CHUNK // L, unroll=8)
        def _(j):
            xv = buf[pl.ds(j * L, L)]
            t = xv * invw + bias
            bi = jnp.minimum(t.astype(jnp.int32), BINS - 1)
            plsc.addupdate_scatter(hist, [bi + lane_off], ones)

    pltpu.async_copy(x_hbm.at[pl.ds(base, CHUNK)], buf_a, sem_a)

    def pair_body(p, _):
        c0 = 2 * p
        pltpu.make_async_copy(
            x_hbm.at[pl.ds(base + c0 * CHUNK, CHUNK)], buf_a, sem_a).wait()
        pltpu.async_copy(
            x_hbm.at[pl.ds(base + (c0 + 1) * CHUNK, CHUNK)], buf_b, sem_b)
        process(buf_a)
        pltpu.make_async_copy(
            x_hbm.at[pl.ds(base + (c0 + 1) * CHUNK, CHUNK)], buf_b, sem_b).wait()

        @pl.when(c0 + 2 < NCH)
        def _():
            pltpu.async_copy(
                x_hbm.at[pl.ds(base + (c0 + 2) * CHUNK, CHUNK)], buf_a, sem_a)

        process(buf_b)
        return 0

    lax.fori_loop(0, NCH // 2, pair_body, 0)

    def red_body(j, _):
        acc = hist[pl.ds(j * L, L)]
        for l in range(1, L):
            acc = acc + hist[pl.ds(l * BINS + j * L, L)]
        red[pl.ds(j * L, L)] = acc
        return 0

    lax.fori_loop(0, BINS // L, red_body, 0)
    pltpu.sync_copy(red, out_hbm.at[pl.ds(wid * BINS, BINS)])


def kernel(x):
    xr = x.reshape(ROWS, COLS)
    mn, mx = _minmax(xr)
    min_val = mn[0, 0]
    max_val = mx[0, 0]

    bin_width = (max_val - min_val) / BINS
    safe_w = jnp.maximum(bin_width, EPS)
    inv_w = 1.0 / safe_w

    min_neg = jnp.minimum(min_val, 0.0)
    max_pos = jnp.maximum(max_val, 0.0)
    scale = jnp.maximum((max_pos - min_neg) / float(Q_MAX - Q_MIN), EPS)
    zero_point = jnp.clip(Q_MIN - jnp.round(min_neg / scale),
                          float(Q_MIN), float(Q_MAX))

    params = jnp.concatenate([
        jnp.broadcast_to(inv_w, (L,)),
        jnp.broadcast_to(-min_val * inv_w, (L,)),
    ]).astype(jnp.float32)
    hist32 = _sc_hist(x, params)
    histogram = _hist_sum(hist32.reshape(NW, BINS)).reshape(BINS)

    qp = jnp.stack([scale, 1.0 / scale, zero_point]).reshape(1, 3)
    out = _fakequant(qp, xr).reshape(N)

    return out, histogram, scale, zero_point.astype(jnp.int32)


# speedup vs baseline: 124.6317x; 1.3840x over previous
import functools

import jax
import jax.numpy as jnp
import numpy as np
from jax import lax
from jax.experimental import pallas as pl
from jax.experimental.pallas import tpu as pltpu
from jax.experimental.pallas import tpu_sc as plsc

BINS = 2048
Q_MIN, Q_MAX = 0, 255
EPS = float(np.finfo(np.float32).eps)

N = 33554432
MM_BLK = N // 16
FQ_BLK = N // 32

NC, NS, L = 2, 16, 16
NW = NC * NS
PER_W = N // NW
CHUNK = 16384
NCH = PER_W // CHUNK
HIST_WORDS = L * BINS


def _mm_body(x_ref, min_ref, max_ref):
    i = pl.program_id(0)
    bmin = jnp.min(x_ref[...])
    bmax = jnp.max(x_ref[...])

    @pl.when(i == 0)
    def _():
        min_ref[0, 0] = bmin
        max_ref[0, 0] = bmax

    @pl.when(i != 0)
    def _():
        min_ref[0, 0] = jnp.minimum(min_ref[0, 0], bmin)
        max_ref[0, 0] = jnp.maximum(max_ref[0, 0], bmax)


_minmax = pl.pallas_call(
    _mm_body,
    grid=(N // MM_BLK,),
    in_specs=[pl.BlockSpec((MM_BLK,), lambda i: (i,))],
    out_specs=[
        pl.BlockSpec((1, 1), lambda i: (0, 0), memory_space=pltpu.SMEM),
        pl.BlockSpec((1, 1), lambda i: (0, 0), memory_space=pltpu.SMEM),
    ],
    out_shape=[
        jax.ShapeDtypeStruct((1, 1), jnp.float32),
        jax.ShapeDtypeStruct((1, 1), jnp.float32),
    ],
)


def _fq_body(qp_ref, x_ref, o_ref):
    s = qp_ref[0, 0]
    inv_s = qp_ref[0, 1]
    z = qp_ref[0, 2]
    q = jnp.clip(jnp.round(x_ref[...] * inv_s) + z, 0.0, 255.0)
    o_ref[...] = (q - z) * s


_fakequant = pl.pallas_call(
    _fq_body,
    grid=(N // FQ_BLK,),
    in_specs=[
        pl.BlockSpec(memory_space=pltpu.SMEM),
        pl.BlockSpec((FQ_BLK,), lambda i: (i,)),
    ],
    out_specs=pl.BlockSpec((FQ_BLK,), lambda i: (i,)),
    out_shape=jax.ShapeDtypeStruct((N,), jnp.float32),
)


def _hsum_body(h_ref, o_ref):
    acc = h_ref[pl.ds(0, BINS)]
    for r in range(1, NW):
        acc = acc + h_ref[pl.ds(r * BINS, BINS)]
    o_ref[...] = acc


_hist_sum = pl.pallas_call(
    _hsum_body,
    out_shape=jax.ShapeDtypeStruct((BINS,), jnp.float32),
)


@functools.partial(
    pl.kernel,
    out_type=jax.ShapeDtypeStruct((NW * BINS,), jnp.float32),
    mesh=plsc.VectorSubcoreMesh(core_axis_name="c", subcore_axis_name="s"),
    compiler_params=pltpu.CompilerParams(needs_layout_passes=False,
                                         use_tc_tiling_on_sc=True),
    scratch_types=[
        pltpu.VMEM((CHUNK,), jnp.float32),
        pltpu.VMEM((CHUNK,), jnp.float32),
        pltpu.VMEM((HIST_WORDS,), jnp.float32),
        pltpu.VMEM((BINS,), jnp.float32),
        pltpu.VMEM((L,), jnp.float32),
        pltpu.VMEM((L,), jnp.float32),
        pltpu.SemaphoreType.DMA,
        pltpu.SemaphoreType.DMA,
    ],
)
def _sc_hist(x_hbm, params_hbm, out_hbm, buf_a, buf_b, hist, red,
             pinvw, pbias, sem_a, sem_b):
    wid = lax.axis_index("s") * NC + lax.axis_index("c")
    base = wid * PER_W

    pltpu.sync_copy(params_hbm.at[pl.ds(0, L)], pinvw)
    pltpu.sync_copy(params_hbm.at[pl.ds(L, L)], pbias)
    invw = pinvw[...]
    bias = pbias[...]
    lane_off = lax.iota(jnp.int32, L) * BINS
    ones = jnp.full((L,), 1.0, jnp.float32)
    zeros = jnp.zeros((L,), jnp.float32)

    def zero_body(j, _):
        hist[pl.ds(j * L, L)] = zeros
        return 0

    lax.fori_loop(0, HIST_WORDS // L, zero_body, 0)

    def process(buf):
        @plsc.parallel_loop(0, CHUNK // L, unroll=8)
        def _(j):
            xv = buf[pl.ds(j * L, L)]
            t = xv * invw + bias
            bi = jnp.minimum(t.astype(jnp.int32), BINS - 1)
            plsc.addupdate_scatter(hist, [bi + lane_off], ones)

    pltpu.async_copy(x_hbm.at[pl.ds(base, CHUNK)], buf_a, sem_a)

    def pair_body(p, _):
        c0 = 2 * p
        pltpu.make_async_copy(
            x_hbm.at[pl.ds(base + c0 * CHUNK, CHUNK)], buf_a, sem_a).wait()
        pltpu.async_copy(
            x_hbm.at[pl.ds(base + (c0 + 1) * CHUNK, CHUNK)], buf_b, sem_b)
        process(buf_a)
        pltpu.make_async_copy(
            x_hbm.at[pl.ds(base + (c0 + 1) * CHUNK, CHUNK)], buf_b, sem_b).wait()

        @pl.when(c0 + 2 < NCH)
        def _():
            pltpu.async_copy(
                x_hbm.at[pl.ds(base + (c0 + 2) * CHUNK, CHUNK)], buf_a, sem_a)

        process(buf_b)
        return 0

    lax.fori_loop(0, NCH // 2, pair_body, 0)

    def red_body(j, _):
        acc = hist[pl.ds(j * L, L)]
        for l in range(1, L):
            acc = acc + hist[pl.ds(l * BINS + j * L, L)]
        red[pl.ds(j * L, L)] = acc
        return 0

    lax.fori_loop(0, BINS // L, red_body, 0)
    pltpu.sync_copy(red, out_hbm.at[pl.ds(wid * BINS, BINS)])


def kernel(x):
    mn, mx = _minmax(x)
    min_val = mn[0, 0]
    max_val = mx[0, 0]

    bin_width = (max_val - min_val) / BINS
    safe_w = jnp.maximum(bin_width, EPS)
    inv_w = 1.0 / safe_w

    min_neg = jnp.minimum(min_val, 0.0)
    max_pos = jnp.maximum(max_val, 0.0)
    scale = jnp.maximum((max_pos - min_neg) / float(Q_MAX - Q_MIN), EPS)
    zero_point = jnp.clip(Q_MIN - jnp.round(min_neg / scale),
                          float(Q_MIN), float(Q_MAX))

    params = jnp.concatenate([
        jnp.broadcast_to(inv_w, (L,)),
        jnp.broadcast_to(-min_val * inv_w, (L,)),
    ]).astype(jnp.float32)
    hist32 = _sc_hist(x, params)
    histogram = _hist_sum(hist32)

    qp = jnp.stack([scale, 1.0 / scale, zero_point]).reshape(1, 3)
    out = _fakequant(qp, x)

    return out, histogram, scale, zero_point.astype(jnp.int32)


# speedup vs baseline: 171.4688x; 1.3758x over previous
import functools

import jax
import jax.numpy as jnp
import numpy as np
from jax import lax
from jax.experimental import pallas as pl
from jax.experimental.pallas import tpu as pltpu
from jax.experimental.pallas import tpu_sc as plsc

BINS = 2048
Q_MIN, Q_MAX = 0, 255
EPS = float(np.finfo(np.float32).eps)

N = 33554432
MM_ROWS = N // 128
MM_BLK = MM_ROWS // 16
FQ_BLK = N // 32

NC, NS, L = 2, 16, 16
NW = NC * NS
PER_W = N // NW
CHUNK = 16384
NCH = PER_W // CHUNK
HIST_WORDS = L * BINS


def _mm_body(x_ref, min_ref, max_ref):
    i = pl.program_id(0)
    bmin = jnp.min(x_ref[...])
    bmax = jnp.max(x_ref[...])

    @pl.when(i == 0)
    def _():
        min_ref[0, 0] = bmin
        max_ref[0, 0] = bmax

    @pl.when(i != 0)
    def _():
        min_ref[0, 0] = jnp.minimum(min_ref[0, 0], bmin)
        max_ref[0, 0] = jnp.maximum(max_ref[0, 0], bmax)


_minmax = pl.pallas_call(
    _mm_body,
    grid=(MM_ROWS // MM_BLK,),
    in_specs=[pl.BlockSpec((MM_BLK, 128), lambda i: (i, 0))],
    out_specs=[
        pl.BlockSpec((1, 1), lambda i: (0, 0), memory_space=pltpu.SMEM),
        pl.BlockSpec((1, 1), lambda i: (0, 0), memory_space=pltpu.SMEM),
    ],
    out_shape=[
        jax.ShapeDtypeStruct((1, 1), jnp.float32),
        jax.ShapeDtypeStruct((1, 1), jnp.float32),
    ],
)


def _fq_body(qp_ref, x_ref, o_ref):
    s = qp_ref[0, 0]
    inv_s = qp_ref[0, 1]
    z = qp_ref[0, 2]
    q = jnp.clip(jnp.round(x_ref[...] * inv_s) + z, 0.0, 255.0)
    o_ref[...] = (q - z) * s


_fakequant = pl.pallas_call(
    _fq_body,
    grid=(N // FQ_BLK,),
    in_specs=[
        pl.BlockSpec(memory_space=pltpu.SMEM),
        pl.BlockSpec((FQ_BLK,), lambda i: (i,)),
    ],
    out_specs=pl.BlockSpec((FQ_BLK,), lambda i: (i,)),
    out_shape=jax.ShapeDtypeStruct((N,), jnp.float32),
)


def _hsum_body(h_ref, o_ref):
    acc = h_ref[pl.ds(0, BINS)]
    for r in range(1, NW):
        acc = acc + h_ref[pl.ds(r * BINS, BINS)]
    o_ref[...] = acc


_hist_sum = pl.pallas_call(
    _hsum_body,
    out_shape=jax.ShapeDtypeStruct((BINS,), jnp.float32),
)


@functools.partial(
    pl.kernel,
    out_type=jax.ShapeDtypeStruct((NW * BINS,), jnp.float32),
    mesh=plsc.VectorSubcoreMesh(core_axis_name="c", subcore_axis_name="s"),
    compiler_params=pltpu.CompilerParams(needs_layout_passes=False,
                                         use_tc_tiling_on_sc=True),
    scratch_types=[
        pltpu.VMEM((CHUNK,), jnp.float32),
        pltpu.VMEM((CHUNK,), jnp.float32),
        pltpu.VMEM((HIST_WORDS,), jnp.float32),
        pltpu.VMEM((BINS,), jnp.float32),
        pltpu.VMEM((L,), jnp.float32),
        pltpu.VMEM((L,), jnp.float32),
        pltpu.SemaphoreType.DMA,
        pltpu.SemaphoreType.DMA,
    ],
)
def _sc_hist(x_hbm, params_hbm, out_hbm, buf_a, buf_b, hist, red,
             pinvw, pbias, sem_a, sem_b):
    wid = lax.axis_index("s") * NC + lax.axis_index("c")
    base = wid * PER_W

    pltpu.sync_copy(params_hbm.at[pl.ds(0, L)], pinvw)
    pltpu.sync_copy(params_hbm.at[pl.ds(L, L)], pbias)
    invw = pinvw[...]
    bias = pbias[...]
    lane_off = lax.iota(jnp.int32, L) * BINS
    ones = jnp.full((L,), 1.0, jnp.float32)
    zeros = jnp.zeros((L,), jnp.float32)

    def zero_body(j, _):
        hist[pl.ds(j * L, L)] = zeros
        return 0

    lax.fori_loop(0, HIST_WORDS // L, zero_body, 0)

    def process(buf):
        @plsc.parallel_loop(0, CHUNK // L, unroll=16)
        def _(j):
            xv = buf[pl.ds(j * L, L)]
            t = xv * invw + bias
            bi = jnp.minimum(t.astype(jnp.int32), BINS - 1)
            plsc.addupdate_scatter(hist, [bi + lane_off], ones)

    pltpu.async_copy(x_hbm.at[pl.ds(base, CHUNK)], buf_a, sem_a)

    def pair_body(p, _):
        c0 = 2 * p
        pltpu.make_async_copy(
            x_hbm.at[pl.ds(base + c0 * CHUNK, CHUNK)], buf_a, sem_a).wait()
        pltpu.async_copy(
            x_hbm.at[pl.ds(base + (c0 + 1) * CHUNK, CHUNK)], buf_b, sem_b)
        process(buf_a)
        pltpu.make_async_copy(
            x_hbm.at[pl.ds(base + (c0 + 1) * CHUNK, CHUNK)], buf_b, sem_b).wait()

        @pl.when(c0 + 2 < NCH)
        def _():
            pltpu.async_copy(
                x_hbm.at[pl.ds(base + (c0 + 2) * CHUNK, CHUNK)], buf_a, sem_a)

        process(buf_b)
        return 0

    lax.fori_loop(0, NCH // 2, pair_body, 0)

    def red_body(j, _):
        acc = hist[pl.ds(j * L, L)]
        for l in range(1, L):
            acc = acc + hist[pl.ds(l * BINS + j * L, L)]
        red[pl.ds(j * L, L)] = acc
        return 0

    lax.fori_loop(0, BINS // L, red_body, 0)
    pltpu.sync_copy(red, out_hbm.at[pl.ds(wid * BINS, BINS)])


def kernel(x):
    mn, mx = _minmax(x.reshape(MM_ROWS, 128))
    min_val = mn[0, 0]
    max_val = mx[0, 0]

    bin_width = (max_val - min_val) / BINS
    safe_w = jnp.maximum(bin_width, EPS)
    inv_w = 1.0 / safe_w

    min_neg = jnp.minimum(min_val, 0.0)
    max_pos = jnp.maximum(max_val, 0.0)
    scale = jnp.maximum((max_pos - min_neg) / float(Q_MAX - Q_MIN), EPS)
    zero_point = jnp.clip(Q_MIN - jnp.round(min_neg / scale),
                          float(Q_MIN), float(Q_MAX))

    params = jnp.concatenate([
        jnp.broadcast_to(inv_w, (L,)),
        jnp.broadcast_to(-min_val * inv_w, (L,)),
    ]).astype(jnp.float32)
    hist32 = _sc_hist(x, params)
    histogram = _hist_sum(hist32)

    qp = jnp.stack([scale, 1.0 / scale, zero_point]).reshape(1, 3)
    out = _fakequant(qp, x)

    return out, histogram, scale, zero_point.astype(jnp.int32)


# speedup vs baseline: 172.6394x; 1.0068x over previous
import functools

import jax
import jax.numpy as jnp
import numpy as np
from jax import lax
from jax.experimental import pallas as pl
from jax.experimental.pallas import tpu as pltpu
from jax.experimental.pallas import tpu_sc as plsc

BINS = 2048
Q_MIN, Q_MAX = 0, 255
EPS = float(np.finfo(np.float32).eps)

N = 33554432
MM_ROWS = N // 128
MM_BLK = MM_ROWS // 16
FQ_BLK = N // 32

NC, NS, L = 2, 16, 16
NW = NC * NS
PER_W = N // NW
CHUNK = 16384
NCH = PER_W // CHUNK
REP_STRIDE = BINS + L + 1
HIST_WORDS = L * REP_STRIDE


def _mm_body(x_ref, min_ref, max_ref):
    i = pl.program_id(0)
    bmin = jnp.min(x_ref[...])
    bmax = jnp.max(x_ref[...])

    @pl.when(i == 0)
    def _():
        min_ref[0, 0] = bmin
        max_ref[0, 0] = bmax

    @pl.when(i != 0)
    def _():
        min_ref[0, 0] = jnp.minimum(min_ref[0, 0], bmin)
        max_ref[0, 0] = jnp.maximum(max_ref[0, 0], bmax)


_minmax = pl.pallas_call(
    _mm_body,
    grid=(MM_ROWS // MM_BLK,),
    in_specs=[pl.BlockSpec((MM_BLK, 128), lambda i: (i, 0))],
    out_specs=[
        pl.BlockSpec((1, 1), lambda i: (0, 0), memory_space=pltpu.SMEM),
        pl.BlockSpec((1, 1), lambda i: (0, 0), memory_space=pltpu.SMEM),
    ],
    out_shape=[
        jax.ShapeDtypeStruct((1, 1), jnp.float32),
        jax.ShapeDtypeStruct((1, 1), jnp.float32),
    ],
)


def _fq_body(qp_ref, x_ref, o_ref):
    s = qp_ref[0, 0]
    inv_s = qp_ref[0, 1]
    z = qp_ref[0, 2]
    q = jnp.clip(jnp.round(x_ref[...] * inv_s) + z, 0.0, 255.0)
    o_ref[...] = (q - z) * s


_fakequant = pl.pallas_call(
    _fq_body,
    grid=(N // FQ_BLK,),
    in_specs=[
        pl.BlockSpec(memory_space=pltpu.SMEM),
        pl.BlockSpec((FQ_BLK,), lambda i: (i,)),
    ],
    out_specs=pl.BlockSpec((FQ_BLK,), lambda i: (i,)),
    out_shape=jax.ShapeDtypeStruct((N,), jnp.float32),
)


def _hsum_body(h_ref, o_ref):
    acc = h_ref[pl.ds(0, BINS)]
    for r in range(1, NW):
        acc = acc + h_ref[pl.ds(r * BINS, BINS)]
    o_ref[...] = acc


_hist_sum = pl.pallas_call(
    _hsum_body,
    out_shape=jax.ShapeDtypeStruct((BINS,), jnp.float32),
)


@functools.partial(
    pl.kernel,
    out_type=jax.ShapeDtypeStruct((NW * BINS,), jnp.float32),
    mesh=plsc.VectorSubcoreMesh(core_axis_name="c", subcore_axis_name="s"),
    compiler_params=pltpu.CompilerParams(needs_layout_passes=False,
                                         use_tc_tiling_on_sc=True),
    scratch_types=[
        pltpu.VMEM((CHUNK,), jnp.float32),
        pltpu.VMEM((CHUNK,), jnp.float32),
        pltpu.VMEM((HIST_WORDS,), jnp.float32),
        pltpu.VMEM((BINS,), jnp.float32),
        pltpu.VMEM((L,), jnp.float32),
        pltpu.VMEM((L,), jnp.float32),
        pltpu.SemaphoreType.DMA,
        pltpu.SemaphoreType.DMA,
    ],
)
def _sc_hist(x_hbm, params_hbm, out_hbm, buf_a, buf_b, hist, red,
             pinvw, pbias, sem_a, sem_b):
    wid = lax.axis_index("s") * NC + lax.axis_index("c")
    base = wid * PER_W

    pltpu.sync_copy(params_hbm.at[pl.ds(0, L)], pinvw)
    pltpu.sync_copy(params_hbm.at[pl.ds(L, L)], pbias)
    invw = pinvw[...]
    bias = pbias[...]
    lane_off = lax.iota(jnp.int32, L) * REP_STRIDE
    ones = jnp.full((L,), 1.0, jnp.float32)
    zeros = jnp.zeros((L,), jnp.float32)

    def zero_body(j, _):
        hist[pl.ds(j * L, L)] = zeros
        return 0

    lax.fori_loop(0, HIST_WORDS // L, zero_body, 0)

    def process(buf):
        @plsc.parallel_loop(0, CHUNK // L, unroll=16)
        def _(j):
            xv = buf[pl.ds(j * L, L)]
            t = xv * invw + bias
            bi = jnp.minimum(t.astype(jnp.int32), BINS - 1)
            plsc.addupdate_scatter(hist, [bi + lane_off], ones)

    pltpu.async_copy(x_hbm.at[pl.ds(base, CHUNK)], buf_a, sem_a)

    def pair_body(p, _):
        c0 = 2 * p
        pltpu.make_async_copy(
            x_hbm.at[pl.ds(base + c0 * CHUNK, CHUNK)], buf_a, sem_a).wait()
        pltpu.async_copy(
            x_hbm.at[pl.ds(base + (c0 + 1) * CHUNK, CHUNK)], buf_b, sem_b)
        process(buf_a)
        pltpu.make_async_copy(
            x_hbm.at[pl.ds(base + (c0 + 1) * CHUNK, CHUNK)], buf_b, sem_b).wait()

        @pl.when(c0 + 2 < NCH)
        def _():
            pltpu.async_copy(
                x_hbm.at[pl.ds(base + (c0 + 2) * CHUNK, CHUNK)], buf_a, sem_a)

        process(buf_b)
        return 0

    lax.fori_loop(0, NCH // 2, pair_body, 0)

    def red_body(j, _):
        acc = hist[pl.ds(j * L, L)]
        for l in range(1, L):
            acc = acc + hist[pl.ds(l * REP_STRIDE + j * L, L)]
        red[pl.ds(j * L, L)] = acc
        return 0

    lax.fori_loop(0, BINS // L, red_body, 0)
    pltpu.sync_copy(red, out_hbm.at[pl.ds(wid * BINS, BINS)])


def kernel(x):
    mn, mx = _minmax(x.reshape(MM_ROWS, 128))
    min_val = mn[0, 0]
    max_val = mx[0, 0]

    bin_width = (max_val - min_val) / BINS
    safe_w = jnp.maximum(bin_width, EPS)
    inv_w = 1.0 / safe_w

    min_neg = jnp.minimum(min_val, 0.0)
    max_pos = jnp.maximum(max_val, 0.0)
    scale = jnp.maximum((max_pos - min_neg) / float(Q_MAX - Q_MIN), EPS)
    zero_point = jnp.clip(Q_MIN - jnp.round(min_neg / scale),
                          float(Q_MIN), float(Q_MAX))

    params = jnp.concatenate([
        jnp.broadcast_to(inv_w, (L,)),
        jnp.broadcast_to(-min_val * inv_w, (L,)),
    ]).astype(jnp.float32)
    hist32 = _sc_hist(x, params)
    histogram = _hist_sum(hist32)

    qp = jnp.stack([scale, 1.0 / scale, zero_point]).reshape(1, 3)
    out = _fakequant(qp, x)

    return out, histogram, scale, zero_point.astype(jnp.int32)


# speedup vs baseline: 187.9538x; 1.0887x over previous
import functools

import jax
import jax.numpy as jnp
import numpy as np
from jax import lax
from jax.experimental import pallas as pl
from jax.experimental.pallas import tpu as pltpu
from jax.experimental.pallas import tpu_sc as plsc

BINS = 2048
Q_MIN, Q_MAX = 0, 255
EPS = float(np.finfo(np.float32).eps)

N = 33554432
MM_ROWS = N // 128
MM_BLK = MM_ROWS // 16
FQ_BLK = N // 32

NC, NS, L = 2, 16, 16
NW = NC * NS
PER_W = N // NW
CHUNK = 16384
NCH = PER_W // CHUNK
REP_STRIDE = BINS + L + 1
HIST_WORDS = L * REP_STRIDE


def _mm_body(x_ref, min_ref, max_ref):
    i = pl.program_id(0)
    bmin = jnp.min(x_ref[...])
    bmax = jnp.max(x_ref[...])

    @pl.when(i == 0)
    def _():
        min_ref[0, 0] = bmin
        max_ref[0, 0] = bmax

    @pl.when(i != 0)
    def _():
        min_ref[0, 0] = jnp.minimum(min_ref[0, 0], bmin)
        max_ref[0, 0] = jnp.maximum(max_ref[0, 0], bmax)


_minmax = pl.pallas_call(
    _mm_body,
    grid=(MM_ROWS // MM_BLK,),
    in_specs=[pl.BlockSpec((MM_BLK, 128), lambda i: (i, 0))],
    out_specs=[
        pl.BlockSpec((1, 1), lambda i: (0, 0), memory_space=pltpu.SMEM),
        pl.BlockSpec((1, 1), lambda i: (0, 0), memory_space=pltpu.SMEM),
    ],
    out_shape=[
        jax.ShapeDtypeStruct((1, 1), jnp.float32),
        jax.ShapeDtypeStruct((1, 1), jnp.float32),
    ],
)


def _fq_body(qp_ref, x_ref, o_ref):
    s = qp_ref[0, 0]
    inv_s = qp_ref[0, 1]
    z = qp_ref[0, 2]
    q = jnp.clip(jnp.round(x_ref[...] * inv_s) + z, 0.0, 255.0)
    o_ref[...] = (q - z) * s


_fakequant = pl.pallas_call(
    _fq_body,
    grid=(N // FQ_BLK,),
    in_specs=[
        pl.BlockSpec(memory_space=pltpu.SMEM),
        pl.BlockSpec((FQ_BLK,), lambda i: (i,)),
    ],
    out_specs=pl.BlockSpec((FQ_BLK,), lambda i: (i,)),
    out_shape=jax.ShapeDtypeStruct((N,), jnp.float32),
)


def _hsum_body(h_ref, o_ref):
    acc = h_ref[pl.ds(0, BINS)]
    for r in range(1, NW):
        acc = acc + h_ref[pl.ds(r * BINS, BINS)]
    o_ref[...] = acc


_hist_sum = pl.pallas_call(
    _hsum_body,
    out_shape=jax.ShapeDtypeStruct((BINS,), jnp.float32),
)


@functools.partial(
    pl.kernel,
    out_type=jax.ShapeDtypeStruct((NW * BINS,), jnp.float32),
    mesh=plsc.VectorSubcoreMesh(core_axis_name="c", subcore_axis_name="s"),
    compiler_params=pltpu.CompilerParams(needs_layout_passes=False,
                                         use_tc_tiling_on_sc=True),
    scratch_types=[
        pltpu.VMEM((CHUNK,), jnp.float32),
        pltpu.VMEM((CHUNK,), jnp.float32),
        pltpu.VMEM((CHUNK,), jnp.float32),
        pltpu.VMEM((CHUNK,), jnp.float32),
        pltpu.VMEM((HIST_WORDS,), jnp.float32),
        pltpu.VMEM((BINS,), jnp.float32),
        pltpu.VMEM((L,), jnp.float32),
        pltpu.VMEM((L,), jnp.float32),
        pltpu.SemaphoreType.DMA,
        pltpu.SemaphoreType.DMA,
        pltpu.SemaphoreType.DMA,
        pltpu.SemaphoreType.DMA,
    ],
)
def _sc_hist(x_hbm, params_hbm, out_hbm, buf_0, buf_1, buf_2, buf_3,
             hist, red, pinvw, pbias, sem_0, sem_1, sem_2, sem_3):
    wid = lax.axis_index("s") * NC + lax.axis_index("c")
    base = wid * PER_W

    pltpu.sync_copy(params_hbm.at[pl.ds(0, L)], pinvw)
    pltpu.sync_copy(params_hbm.at[pl.ds(L, L)], pbias)
    invw = pinvw[...]
    bias = pbias[...]
    lane_off = lax.iota(jnp.int32, L) * REP_STRIDE
    ones = jnp.full((L,), 1.0, jnp.float32)
    zeros = jnp.zeros((L,), jnp.float32)

    def zero_body(j, _):
        hist[pl.ds(j * L, L)] = zeros
        return 0

    lax.fori_loop(0, HIST_WORDS // L, zero_body, 0)

    def process(buf):
        @plsc.parallel_loop(0, CHUNK // L, unroll=8)
        def _(j):
            xv = buf[pl.ds(j * L, L)]
            t = xv * invw + bias
            bi = jnp.minimum(t.astype(jnp.int32), BINS - 1)
            plsc.addupdate_scatter(hist, [bi + lane_off], ones)

    bufs = (buf_0, buf_1, buf_2, buf_3)
    sems = (sem_0, sem_1, sem_2, sem_3)
    for c in range(3):
        pltpu.async_copy(
            x_hbm.at[pl.ds(base + c * CHUNK, CHUNK)], bufs[c], sems[c])

    def quad_body(q, _):
        c0 = 4 * q
        for b in range(4):
            c = c0 + b
            pltpu.make_async_copy(
                x_hbm.at[pl.ds(base + c * CHUNK, CHUNK)],
                bufs[b], sems[b]).wait()
            nb = (b + 3) % 4

            @pl.when(c + 3 < NCH)
            def _():
                pltpu.async_copy(
                    x_hbm.at[pl.ds(base + (c + 3) * CHUNK, CHUNK)],
                    bufs[nb], sems[nb])

            process(bufs[b])
        return 0

    lax.fori_loop(0, NCH // 4, quad_body, 0)

    def red_body(j, _):
        acc = hist[pl.ds(j * L, L)]
        for l in range(1, L):
            acc = acc + hist[pl.ds(l * REP_STRIDE + j * L, L)]
        red[pl.ds(j * L, L)] = acc
        return 0

    lax.fori_loop(0, BINS // L, red_body, 0)
    pltpu.sync_copy(red, out_hbm.at[pl.ds(wid * BINS, BINS)])


def kernel(x):
    mn, mx = _minmax(x.reshape(MM_ROWS, 128))
    min_val = mn[0, 0]
    max_val = mx[0, 0]

    bin_width = (max_val - min_val) / BINS
    safe_w = jnp.maximum(bin_width, EPS)
    inv_w = 1.0 / safe_w

    min_neg = jnp.minimum(min_val, 0.0)
    max_pos = jnp.maximum(max_val, 0.0)
    scale = jnp.maximum((max_pos - min_neg) / float(Q_MAX - Q_MIN), EPS)
    zero_point = jnp.clip(Q_MIN - jnp.round(min_neg / scale),
                          float(Q_MIN), float(Q_MAX))

    params = jnp.concatenate([
        jnp.broadcast_to(inv_w, (L,)),
        jnp.broadcast_to(-min_val * inv_w, (L,)),
    ]).astype(jnp.float32)
    hist32 = _sc_hist(x, params)
    histogram = _hist_sum(hist32)

    qp = jnp.stack([scale, 1.0 / scale, zero_point]).reshape(1, 3)
    out = _fakequant(qp, x)

    return out, histogram, scale, zero_point.astype(jnp.int32)
